# Initial kernel scaffold; baseline (speedup 1.0000x reference)
import jax
import jax.numpy as jnp
from jax.experimental import pallas as pl


def kernel(x, edge_index, W0, a_src0, a_dst0, b0, W1, a_src1, a_dst1, b1, W2, a_src2, a_dst2, b2):
    raise NotImplementedError("write your pallas kernel here")


# speedup vs baseline: 21.0211x; 21.0211x over previous
import functools

import jax
import jax.numpy as jnp
from jax import lax
from jax.experimental import pallas as pl
from jax.experimental.pallas import tpu as pltpu
from jax.experimental.pallas import tpu_sc as plsc

N = 10000
D = 128
E = 320000
NC = 2
NS = 16
NW = NC * NS
EPT = E // NW
CH = 80
NCHUNK = EPT // CH
RPS = N // NS

ROWB = 2000
GRID = N // ROWB


def _tc_first_body(x_ref, w_ref, asrc_ref, adst_ref, h_ref, asad_ref):
    h = jnp.dot(x_ref[...], w_ref[...], preferred_element_type=jnp.float32)
    h_ref[...] = h
    a_s = jnp.sum(h * asrc_ref[...], axis=1, keepdims=True)
    a_d = jnp.sum(h * adst_ref[...], axis=1, keepdims=True)
    asad_ref[...] = jnp.concatenate([a_s, a_d], axis=1)


def _tc_mid_body(acc_ref, den_ref, b_ref, w_ref, asrc_ref, adst_ref,
                 h_ref, asad_ref):
    num = acc_ref[0] + acc_ref[1]
    den = den_ref[0] + den_ref[1]
    x = num / (den + 1e-16) + b_ref[...]
    x = jnp.where(x >= 0.0, x, 0.1 * x)
    h = jnp.dot(x, w_ref[...], preferred_element_type=jnp.float32)
    h_ref[...] = h
    a_s = jnp.sum(h * asrc_ref[...], axis=1, keepdims=True)
    a_d = jnp.sum(h * adst_ref[...], axis=1, keepdims=True)
    asad_ref[...] = jnp.concatenate([a_s, a_d], axis=1)


def _tc_final_body(acc_ref, den_ref, b_ref, out_ref):
    num = acc_ref[0] + acc_ref[1]
    den = den_ref[0] + den_ref[1]
    x = num / (den + 1e-16) + b_ref[...]
    out_ref[...] = jnp.where(x >= 0.0, x, 0.1 * x)


_full = lambda *shape: pl.BlockSpec(shape, lambda i: (0,) * len(shape))


def _tc_first(x, W, asrc, adst):
    return pl.pallas_call(
        _tc_first_body,
        grid=(GRID,),
        in_specs=[
            pl.BlockSpec((ROWB, D), lambda i: (i, 0)),
            _full(D, D),
            _full(1, D),
            _full(1, D),
        ],
        out_specs=[
            pl.BlockSpec((ROWB, D), lambda i: (i, 0)),
            pl.BlockSpec((ROWB, 2), lambda i: (i, 0)),
        ],
        out_shape=[
            jax.ShapeDtypeStruct((N, D), jnp.float32),
            jax.ShapeDtypeStruct((N, 2), jnp.float32),
        ],
    )(x, W, asrc, adst)


def _tc_mid(acc, den, b, W, asrc, adst):
    return pl.pallas_call(
        _tc_mid_body,
        grid=(GRID,),
        in_specs=[
            pl.BlockSpec((NC, ROWB, D), lambda i: (0, i, 0)),
            pl.BlockSpec((NC, ROWB, 1), lambda i: (0, i, 0)),
            _full(1, D),
            _full(D, D),
            _full(1, D),
            _full(1, D),
        ],
        out_specs=[
            pl.BlockSpec((ROWB, D), lambda i: (i, 0)),
            pl.BlockSpec((ROWB, 2), lambda i: (i, 0)),
        ],
        out_shape=[
            jax.ShapeDtypeStruct((N, D), jnp.float32),
            jax.ShapeDtypeStruct((N, 2), jnp.float32),
        ],
    )(acc, den, b, W, asrc, adst)


def _tc_final(acc, den, b):
    return pl.pallas_call(
        _tc_final_body,
        grid=(GRID,),
        in_specs=[
            pl.BlockSpec((NC, ROWB, D), lambda i: (0, i, 0)),
            pl.BlockSpec((NC, ROWB, 1), lambda i: (0, i, 0)),
            _full(1, D),
        ],
        out_specs=pl.BlockSpec((ROWB, D), lambda i: (i, 0)),
        out_shape=jax.ShapeDtypeStruct((N, D), jnp.float32),
    )(acc, den, b)


_mesh = plsc.VectorSubcoreMesh(core_axis_name="c", subcore_axis_name="s")


@functools.partial(
    pl.kernel,
    out_type=(
        jax.ShapeDtypeStruct((NC, N, D), jnp.float32),
        jax.ShapeDtypeStruct((NC * N,), jnp.float32),
    ),
    mesh=_mesh,
    compiler_params=pltpu.CompilerParams(needs_layout_passes=False),
    scratch_types=[
        pltpu.VMEM((2 * N,), jnp.float32),
        pltpu.VMEM((CH,), jnp.int32),
        pltpu.VMEM((CH,), jnp.int32),
        pltpu.VMEM((CH,), jnp.float32),
        pltpu.VMEM((CH, D), jnp.float32),
        pltpu.VMEM((1000,), jnp.float32),
        pltpu.VMEM_SHARED((N, D), jnp.float32),
        pltpu.VMEM_SHARED((N,), jnp.float32),
        pltpu.SemaphoreType.DMA,
    ],
)
def _sc_edges(src_hbm, dst_hbm, asad_hbm, h_hbm, acc_out, den_out,
              asad_v, src_v, dst_v, w_v, rows_v, zden_v,
              acc_sh, den_sh, sem):
    c = lax.axis_index("c")
    s = lax.axis_index("s")
    z16 = jnp.zeros((16,), jnp.float32)

    def _zrow(i, _):
        for k in range(D // 16):
            rows_v[i, pl.ds(k * 16, 16)] = z16
        return _
    lax.fori_loop(0, 40, _zrow, 0)

    def _zden(i, _):
        zden_v[pl.ds(i * 16, 16)] = z16
        return _
    lax.fori_loop(0, 62, _zden, 0)
    zden_v[pl.ds(984, 16)] = z16

    @pl.when(s < 10)
    def _():
        for k in range(25):
            pltpu.sync_copy(rows_v.at[pl.ds(0, 40)],
                            acc_sh.at[pl.ds(s * 1000 + k * 40, 40)])
        pltpu.sync_copy(zden_v, den_sh.at[pl.ds(s * 1000, 1000)])

    pltpu.sync_copy(asad_hbm, asad_v)

    plsc.subcore_barrier()

    base = (c * NS + s) * EPT

    def _chunk(j, _):
        off = base + j * CH
        pltpu.sync_copy(src_hbm.at[pl.ds(off, CH)], src_v)
        pltpu.sync_copy(dst_hbm.at[pl.ds(off, CH)], dst_v)
        pltpu.async_copy(h_hbm.at[src_v], rows_v, sem).wait()
        for g in range(CH // 16):
            sidx = src_v[pl.ds(g * 16, 16)]
            didx = dst_v[pl.ds(g * 16, 16)]
            av = plsc.load_gather(asad_v, [sidx * 2])
            dv = plsc.load_gather(asad_v, [didx * 2 + 1])
            e = av + dv
            e = jnp.where(e >= 0.0, e, 0.2 * e)
            w_v[pl.ds(g * 16, 16)] = jnp.exp(e)

        def _scale(i, _):
            ws = plsc.load_gather(w_v, [jnp.full((16,), i, jnp.int32)])
            for k in range(D // 16):
                rows_v[i, pl.ds(k * 16, 16)] = rows_v[i, pl.ds(k * 16, 16)] * ws
            return _
        lax.fori_loop(0, CH, _scale, 0)

        pltpu.sync_copy(rows_v, acc_sh.at[dst_v], add=True)
        pltpu.sync_copy(w_v, den_sh.at[dst_v], add=True)
        return _

    lax.fori_loop(0, NCHUNK, _chunk, 0)

    plsc.subcore_barrier()

    @pl.when(s < 10)
    def _():
        for k in range(25):
            pltpu.sync_copy(acc_sh.at[pl.ds(s * 1000 + k * 40, 40)],
                            rows_v.at[pl.ds(0, 40)])
            pltpu.sync_copy(rows_v.at[pl.ds(0, 40)],
                            acc_out.at[c, pl.ds(s * 1000 + k * 40, 40)])
        pltpu.sync_copy(den_sh.at[pl.ds(s * 1000, 1000)], zden_v)
        pltpu.sync_copy(zden_v, den_out.at[pl.ds(c * N + s * 1000, 1000)])


def kernel(x, edge_index, W0, a_src0, a_dst0, b0, W1, a_src1, a_dst1, b1,
           W2, a_src2, a_dst2, b2):
    src = edge_index[0].astype(jnp.int32)
    dst = edge_index[1].astype(jnp.int32)

    r = lambda v: v.reshape(1, D)

    h, asad = _tc_first(x, W0, r(a_src0), r(a_dst0))
    acc, den = _sc_edges(src, dst, asad.reshape(2 * N), h)

    h, asad = _tc_mid(acc, den.reshape(NC, N, 1), r(b0), W1, r(a_src1), r(a_dst1))
    acc, den = _sc_edges(src, dst, asad.reshape(2 * N), h)

    h, asad = _tc_mid(acc, den.reshape(NC, N, 1), r(b1), W2, r(a_src2), r(a_dst2))
    acc, den = _sc_edges(src, dst, asad.reshape(2 * N), h)

    return _tc_final(acc, den.reshape(NC, N, 1), r(b2))


# speedup vs baseline: 38.7000x; 1.8410x over previous
import functools

import jax
import jax.numpy as jnp
from jax import lax
from jax.experimental import pallas as pl
from jax.experimental.pallas import tpu as pltpu
from jax.experimental.pallas import tpu_sc as plsc

N = 10000
D = 128
E = 320000
NC = 2
NS = 16
NW = NC * NS
EPT = E // NW
CH = 80
NCHUNK = EPT // CH

ROWB = 2000
GRID = N // ROWB


def _tc_first_body(x_ref, w_ref, asrc_ref, adst_ref, h_ref, asad_ref):
    h = jnp.dot(x_ref[...], w_ref[...], preferred_element_type=jnp.float32)
    h_ref[...] = h
    a_s = jnp.sum(h * asrc_ref[...], axis=1, keepdims=True)
    a_d = jnp.sum(h * adst_ref[...], axis=1, keepdims=True)
    asad_ref[...] = jnp.concatenate([a_s, a_d], axis=1)


def _tc_mid_body(acc_ref, den_ref, b_ref, w_ref, asrc_ref, adst_ref,
                 h_ref, asad_ref):
    num = acc_ref[0] + acc_ref[1]
    den = den_ref[0] + den_ref[1]
    x = num / (den + 1e-16) + b_ref[...]
    x = jnp.where(x >= 0.0, x, 0.1 * x)
    h = jnp.dot(x, w_ref[...], preferred_element_type=jnp.float32)
    h_ref[...] = h
    a_s = jnp.sum(h * asrc_ref[...], axis=1, keepdims=True)
    a_d = jnp.sum(h * adst_ref[...], axis=1, keepdims=True)
    asad_ref[...] = jnp.concatenate([a_s, a_d], axis=1)


def _tc_final_body(acc_ref, den_ref, b_ref, out_ref):
    num = acc_ref[0] + acc_ref[1]
    den = den_ref[0] + den_ref[1]
    x = num / (den + 1e-16) + b_ref[...]
    out_ref[...] = jnp.where(x >= 0.0, x, 0.1 * x)


_full = lambda *shape: pl.BlockSpec(shape, lambda i: (0,) * len(shape))


def _tc_first(x, W, asrc, adst):
    return pl.pallas_call(
        _tc_first_body,
        grid=(GRID,),
        in_specs=[
            pl.BlockSpec((ROWB, D), lambda i: (i, 0)),
            _full(D, D),
            _full(1, D),
            _full(1, D),
        ],
        out_specs=[
            pl.BlockSpec((ROWB, D), lambda i: (i, 0)),
            pl.BlockSpec((ROWB, 2), lambda i: (i, 0)),
        ],
        out_shape=[
            jax.ShapeDtypeStruct((N, D), jnp.float32),
            jax.ShapeDtypeStruct((N, 2), jnp.float32),
        ],
    )(x, W, asrc, adst)


def _tc_mid(acc, den, b, W, asrc, adst):
    return pl.pallas_call(
        _tc_mid_body,
        grid=(GRID,),
        in_specs=[
            pl.BlockSpec((NC, ROWB, D), lambda i: (0, i, 0)),
            pl.BlockSpec((NC, ROWB, 1), lambda i: (0, i, 0)),
            _full(1, D),
            _full(D, D),
            _full(1, D),
            _full(1, D),
        ],
        out_specs=[
            pl.BlockSpec((ROWB, D), lambda i: (i, 0)),
            pl.BlockSpec((ROWB, 2), lambda i: (i, 0)),
        ],
        out_shape=[
            jax.ShapeDtypeStruct((N, D), jnp.float32),
            jax.ShapeDtypeStruct((N, 2), jnp.float32),
        ],
    )(acc, den, b, W, asrc, adst)


def _tc_final(acc, den, b):
    return pl.pallas_call(
        _tc_final_body,
        grid=(GRID,),
        in_specs=[
            pl.BlockSpec((NC, ROWB, D), lambda i: (0, i, 0)),
            pl.BlockSpec((NC, ROWB, 1), lambda i: (0, i, 0)),
            _full(1, D),
        ],
        out_specs=pl.BlockSpec((ROWB, D), lambda i: (i, 0)),
        out_shape=jax.ShapeDtypeStruct((N, D), jnp.float32),
    )(acc, den, b)


_mesh = plsc.VectorSubcoreMesh(core_axis_name="c", subcore_axis_name="s")


@functools.partial(
    pl.kernel,
    out_type=(
        jax.ShapeDtypeStruct((NC, N, D), jnp.float32),
        jax.ShapeDtypeStruct((NC * N,), jnp.float32),
    ),
    mesh=_mesh,
    compiler_params=pltpu.CompilerParams(needs_layout_passes=False),
    scratch_types=[
        pltpu.VMEM((2 * N,), jnp.float32),
        pltpu.VMEM((2, CH), jnp.int32),
        pltpu.VMEM((2, CH), jnp.int32),
        pltpu.VMEM((CH,), jnp.float32),
        pltpu.VMEM((CH,), jnp.float32),
        pltpu.VMEM((CH, D), jnp.float32),
        pltpu.VMEM((CH, D), jnp.float32),
        pltpu.VMEM((1000,), jnp.float32),
        pltpu.VMEM_SHARED((N, D), jnp.float32),
        pltpu.VMEM_SHARED((N,), jnp.float32),
        pltpu.SemaphoreType.DMA,
        pltpu.SemaphoreType.DMA,
        pltpu.SemaphoreType.DMA,
        pltpu.SemaphoreType.DMA,
        pltpu.SemaphoreType.DMA,
    ],
)
def _sc_edges(ei_hbm, asad_hbm, h_hbm, acc_out, den_out,
              asad_v, eib0, eib1, wb0, wb1, rowsb0, rowsb1, zden_v,
              acc_sh, den_sh, sem_g0, sem_g1, sem_s0, sem_s1, sem_wb):
    c = lax.axis_index("c")
    s = lax.axis_index("s")
    z16 = jnp.zeros((16,), jnp.float32)
    eib = (eib0, eib1)
    wb = (wb0, wb1)
    rows = (rowsb0, rowsb1)
    sem_g = (sem_g0, sem_g1)
    sem_s = (sem_s0, sem_s1)

    def _zrow(i, _):
        for k in range(D // 16):
            rowsb0[i, pl.ds(k * 16, 16)] = z16
        return _
    lax.fori_loop(0, 40, _zrow, 0)

    def _zden(i, _):
        zden_v[pl.ds(i * 16, 16)] = z16
        return _
    lax.fori_loop(0, 62, _zden, 0)
    zden_v[pl.ds(984, 16)] = z16

    @pl.when(s < 10)
    def _():
        zcps = [
            pltpu.async_copy(rowsb0.at[pl.ds(0, 40)],
                             acc_sh.at[pl.ds(s * 1000 + k * 40, 40)], sem_wb)
            for k in range(25)
        ]
        for cp in zcps:
            cp.wait()
        pltpu.sync_copy(zden_v, den_sh.at[pl.ds(s * 1000, 1000)])

    pltpu.sync_copy(asad_hbm, asad_v)

    plsc.subcore_barrier()

    cbase = (c * NS + s) * NCHUNK

    def _compute(b):
        for g in range(CH // 16):
            sidx = eib[b][0, pl.ds(g * 16, 16)]
            didx = eib[b][1, pl.ds(g * 16, 16)]
            av = plsc.load_gather(asad_v, [sidx * 2])
            dv = plsc.load_gather(asad_v, [didx * 2 + 1])
            e = av + dv
            e = jnp.where(e >= 0.0, e, 0.2 * e)
            wb[b][pl.ds(g * 16, 16)] = jnp.exp(e)

        @plsc.parallel_loop(0, CH, 1, unroll=8)
        def _(i):
            ws = plsc.load_gather(wb[b], [jnp.full((16,), i, jnp.int32)])
            for k in range(D // 16):
                rows[b][i, pl.ds(k * 16, 16)] = rows[b][i, pl.ds(k * 16, 16)] * ws

    def _issue_scatter(b):
        pltpu.async_copy(rows[b], acc_sh.at[eib[b].at[1]], sem_s[b], add=True)
        pltpu.async_copy(wb[b], den_sh.at[eib[b].at[1]], sem_s[b], add=True)

    def _wait_scatter(b):
        pltpu.make_async_copy(h_hbm.at[pl.ds(0, CH)], rows[b], sem_s[b]).wait()
        pltpu.make_async_copy(den_out.at[pl.ds(0, CH)], wb[b], sem_s[b]).wait()

    def _wait_gather(b):
        pltpu.make_async_copy(h_hbm.at[pl.ds(0, CH)], rows[b], sem_g[b]).wait()

    pltpu.sync_copy(ei_hbm.at[cbase], eib0)
    pltpu.async_copy(h_hbm.at[eib0.at[0]], rowsb0, sem_g0)

    def _pair(p, carry):
        for b in range(2):
            j = 2 * p + b
            _wait_gather(b)
            nb = 1 - b
            if b == 0:
                @pl.when(p > 0)
                def _ws():
                    _wait_scatter(nb)
            else:
                _wait_scatter(nb)
            pltpu.sync_copy(ei_hbm.at[cbase + j + 1], eib[nb])
            pltpu.async_copy(h_hbm.at[eib[nb].at[0]], rows[nb], sem_g[nb])
            _compute(b)
            _issue_scatter(b)
        return carry

    lax.fori_loop(0, (NCHUNK - 1) // 2, _pair, 0)

    _wait_gather(0)
    _wait_scatter(1)
    _compute(0)
    _issue_scatter(0)
    _wait_scatter(0)

    plsc.subcore_barrier()

    @pl.when(s < 10)
    def _():
        prev = [None, None]
        for k in range(25):
            b = k % 2
            if prev[b] is not None:
                prev[b].wait()
            pltpu.sync_copy(acc_sh.at[pl.ds(s * 1000 + k * 40, 40)],
                            rows[b].at[pl.ds(0, 40)])
            prev[b] = pltpu.async_copy(
                rows[b].at[pl.ds(0, 40)],
                acc_out.at[c, pl.ds(s * 1000 + k * 40, 40)], sem_wb)
        prev[0].wait()
        prev[1].wait()
        pltpu.sync_copy(den_sh.at[pl.ds(s * 1000, 1000)], zden_v)
        pltpu.sync_copy(zden_v, den_out.at[pl.ds(c * N + s * 1000, 1000)])


def kernel(x, edge_index, W0, a_src0, a_dst0, b0, W1, a_src1, a_dst1, b1,
           W2, a_src2, a_dst2, b2):
    ei = edge_index.astype(jnp.int32).reshape(2, NW * NCHUNK, CH)
    ei = ei.transpose(1, 0, 2)

    r = lambda v: v.reshape(1, D)

    h, asad = _tc_first(x, W0, r(a_src0), r(a_dst0))
    acc, den = _sc_edges(ei, asad.reshape(2 * N), h)

    h, asad = _tc_mid(acc, den.reshape(NC, N, 1), r(b0), W1, r(a_src1), r(a_dst1))
    acc, den = _sc_edges(ei, asad.reshape(2 * N), h)

    h, asad = _tc_mid(acc, den.reshape(NC, N, 1), r(b1), W2, r(a_src2), r(a_dst2))
    acc, den = _sc_edges(ei, asad.reshape(2 * N), h)

    return _tc_final(acc, den.reshape(NC, N, 1), r(b2))


# speedup vs baseline: 49.9198x; 1.2899x over previous
import functools

import jax
import jax.numpy as jnp
from jax import lax
from jax.experimental import pallas as pl
from jax.experimental.pallas import tpu as pltpu
from jax.experimental.pallas import tpu_sc as plsc

N = 10000
D = 128
E = 320000
NC = 2
NS = 16
NW = NC * NS
EPT = E // NW
CH = 80
NCHUNK = EPT // CH

ROWB = 2000
GRID = N // ROWB


def _tc_first_body(x_ref, w_ref, asrc_ref, adst_ref, h_ref, asad_ref):
    h = jnp.dot(x_ref[...], w_ref[...], preferred_element_type=jnp.float32)
    h_ref[...] = h
    a_s = jnp.sum(h * asrc_ref[...], axis=1, keepdims=True)
    a_d = jnp.sum(h * adst_ref[...], axis=1, keepdims=True)
    asad_ref[...] = jnp.concatenate([a_s, a_d], axis=1)


def _tc_mid_body(acc_ref, den_ref, b_ref, w_ref, asrc_ref, adst_ref,
                 h_ref, asad_ref):
    num = acc_ref[0] + acc_ref[1]
    den = den_ref[0] + den_ref[1]
    x = num / (den + 1e-16) + b_ref[...]
    x = jnp.where(x >= 0.0, x, 0.1 * x)
    h = jnp.dot(x, w_ref[...], preferred_element_type=jnp.float32)
    h_ref[...] = h
    a_s = jnp.sum(h * asrc_ref[...], axis=1, keepdims=True)
    a_d = jnp.sum(h * adst_ref[...], axis=1, keepdims=True)
    asad_ref[...] = jnp.concatenate([a_s, a_d], axis=1)


def _tc_final_body(acc_ref, den_ref, b_ref, out_ref):
    num = acc_ref[0] + acc_ref[1]
    den = den_ref[0] + den_ref[1]
    x = num / (den + 1e-16) + b_ref[...]
    out_ref[...] = jnp.where(x >= 0.0, x, 0.1 * x)


_full = lambda *shape: pl.BlockSpec(shape, lambda i: (0,) * len(shape))


def _tc_first(x, W, asrc, adst):
    return pl.pallas_call(
        _tc_first_body,
        grid=(GRID,),
        in_specs=[
            pl.BlockSpec((ROWB, D), lambda i: (i, 0)),
            _full(D, D),
            _full(1, D),
            _full(1, D),
        ],
        out_specs=[
            pl.BlockSpec((ROWB, D), lambda i: (i, 0)),
            pl.BlockSpec((ROWB, 2), lambda i: (i, 0)),
        ],
        out_shape=[
            jax.ShapeDtypeStruct((N, D), jnp.float32),
            jax.ShapeDtypeStruct((N, 2), jnp.float32),
        ],
    )(x, W, asrc, adst)


def _tc_mid(acc, den, b, W, asrc, adst):
    return pl.pallas_call(
        _tc_mid_body,
        grid=(GRID,),
        in_specs=[
            pl.BlockSpec((NC, ROWB, D), lambda i: (0, i, 0)),
            pl.BlockSpec((NC, ROWB, 1), lambda i: (0, i, 0)),
            _full(1, D),
            _full(D, D),
            _full(1, D),
            _full(1, D),
        ],
        out_specs=[
            pl.BlockSpec((ROWB, D), lambda i: (i, 0)),
            pl.BlockSpec((ROWB, 2), lambda i: (i, 0)),
        ],
        out_shape=[
            jax.ShapeDtypeStruct((N, D), jnp.float32),
            jax.ShapeDtypeStruct((N, 2), jnp.float32),
        ],
    )(acc, den, b, W, asrc, adst)


def _tc_final(acc, den, b):
    return pl.pallas_call(
        _tc_final_body,
        grid=(GRID,),
        in_specs=[
            pl.BlockSpec((NC, ROWB, D), lambda i: (0, i, 0)),
            pl.BlockSpec((NC, ROWB, 1), lambda i: (0, i, 0)),
            _full(1, D),
        ],
        out_specs=pl.BlockSpec((ROWB, D), lambda i: (i, 0)),
        out_shape=jax.ShapeDtypeStruct((N, D), jnp.float32),
    )(acc, den, b)


_mesh = plsc.VectorSubcoreMesh(core_axis_name="c", subcore_axis_name="s")


@functools.partial(
    pl.kernel,
    out_type=(
        jax.ShapeDtypeStruct((NC, N, D), jnp.float32),
        jax.ShapeDtypeStruct((NC * N,), jnp.float32),
    ),
    mesh=_mesh,
    compiler_params=pltpu.CompilerParams(needs_layout_passes=False),
    scratch_types=[
        pltpu.VMEM((2 * N,), jnp.float32),
        pltpu.VMEM((2, CH), jnp.int32),
        pltpu.VMEM((2, CH), jnp.int32),
        pltpu.VMEM((2, CH), jnp.int32),
        pltpu.VMEM((2, CH), jnp.int32),
        pltpu.VMEM((CH,), jnp.float32),
        pltpu.VMEM((CH,), jnp.float32),
        pltpu.VMEM((CH, D), jnp.float32),
        pltpu.VMEM((CH, D), jnp.float32),
        pltpu.VMEM((1000,), jnp.float32),
        pltpu.VMEM_SHARED((N, D), jnp.float32),
        pltpu.VMEM_SHARED((N,), jnp.float32),
        pltpu.SemaphoreType.DMA,
        pltpu.SemaphoreType.DMA,
        pltpu.SemaphoreType.DMA,
        pltpu.SemaphoreType.DMA,
        pltpu.SemaphoreType.DMA,
        pltpu.SemaphoreType.DMA,
        pltpu.SemaphoreType.DMA,
        pltpu.SemaphoreType.DMA,
        pltpu.SemaphoreType.DMA,
    ],
)
def _sc_edges(ei_hbm, asad_hbm, h_hbm, acc_out, den_out,
              asad_v, eib0, eib1, eib2, eib3, wb0, wb1, rowsb0, rowsb1,
              zden_v, acc_sh, den_sh,
              sem_g0, sem_g1, sem_s0, sem_s1,
              sem_e0, sem_e1, sem_e2, sem_e3, sem_wb):
    c = lax.axis_index("c")
    s = lax.axis_index("s")
    z16 = jnp.zeros((16,), jnp.float32)
    eib = (eib0, eib1, eib2, eib3)
    wb = (wb0, wb1)
    rows = (rowsb0, rowsb1)
    sem_g = (sem_g0, sem_g1)
    sem_s = (sem_s0, sem_s1)
    sem_e = (sem_e0, sem_e1, sem_e2, sem_e3)

    def _zrow(i, _):
        for k in range(D // 16):
            rowsb0[i, pl.ds(k * 16, 16)] = z16
        return _
    lax.fori_loop(0, 40, _zrow, 0)

    def _zden(i, _):
        zden_v[pl.ds(i * 16, 16)] = z16
        return _
    lax.fori_loop(0, 62, _zden, 0)
    zden_v[pl.ds(984, 16)] = z16

    @pl.when(s < 10)
    def _():
        zcps = [
            pltpu.async_copy(rowsb0.at[pl.ds(0, 40)],
                             acc_sh.at[pl.ds(s * 1000 + k * 40, 40)], sem_wb)
            for k in range(25)
        ]
        for cp in zcps:
            cp.wait()
        pltpu.sync_copy(zden_v, den_sh.at[pl.ds(s * 1000, 1000)])

    pltpu.sync_copy(asad_hbm, asad_v)

    plsc.subcore_barrier()

    cbase = (c * NS + s) * NCHUNK

    def _compute(b, rb):
        for g in range(CH // 16):
            sidx = eib[b][0, pl.ds(g * 16, 16)]
            didx = eib[b][1, pl.ds(g * 16, 16)]
            av = plsc.load_gather(asad_v, [sidx * 2])
            dv = plsc.load_gather(asad_v, [didx * 2 + 1])
            e = av + dv
            e = jnp.where(e >= 0.0, e, 0.2 * e)
            wb[rb][pl.ds(g * 16, 16)] = jnp.exp(e)

        @plsc.parallel_loop(0, CH, 1, unroll=8)
        def _(i):
            ws = plsc.load_gather(wb[rb], [jnp.full((16,), i, jnp.int32)])
            for k in range(D // 16):
                rows[rb][i, pl.ds(k * 16, 16)] = (
                    rows[rb][i, pl.ds(k * 16, 16)] * ws)

    def _issue_scatter(b, rb):
        pltpu.async_copy(rows[rb], acc_sh.at[eib[b].at[1]], sem_s[rb], add=True)
        pltpu.async_copy(wb[rb], den_sh.at[eib[b].at[1]], sem_s[rb], add=True)

    def _wait_scatter(b):
        pltpu.make_async_copy(h_hbm.at[pl.ds(0, CH)], rows[b], sem_s[b]).wait()
        pltpu.make_async_copy(den_out.at[pl.ds(0, CH)], wb[b], sem_s[b]).wait()

    def _wait_gather(b):
        pltpu.make_async_copy(h_hbm.at[pl.ds(0, CH)], rows[b], sem_g[b]).wait()

    def _wait_ei(e):
        pltpu.make_async_copy(ei_hbm.at[cbase], eib[e], sem_e[e]).wait()

    pltpu.sync_copy(ei_hbm.at[cbase], eib0)
    pltpu.async_copy(h_hbm.at[eib0.at[0]], rowsb0, sem_g0)
    pltpu.async_copy(ei_hbm.at[cbase + 1], eib1, sem_e1)

    def _quad(q, carry):
        for b in range(4):
            j = 4 * q + b
            rb = b % 2
            _wait_gather(rb)
            if b == 0:
                @pl.when(q > 0)
                def _ws():
                    _wait_scatter(1)
            else:
                _wait_scatter(1 - rb)
            _wait_ei((b + 1) % 4)
            pltpu.async_copy(h_hbm.at[eib[(b + 1) % 4].at[0]],
                             rows[1 - rb], sem_g[1 - rb])
            @pl.when(j + 2 < NCHUNK)
            def _pe():
                pltpu.async_copy(ei_hbm.at[cbase + j + 2],
                                 eib[(b + 2) % 4], sem_e[(b + 2) % 4])
            _compute(b, rb)
            _issue_scatter(b, rb)
        return carry

    lax.fori_loop(0, NCHUNK // 4, _quad, 0)

    _wait_gather(0)
    _wait_scatter(1)
    _compute(0, 0)
    _issue_scatter(0, 0)
    _wait_scatter(0)

    plsc.subcore_barrier()

    @pl.when(s < 10)
    def _():
        prev = [None, None]
        for k in range(25):
            b = k % 2
            if prev[b] is not None:
                prev[b].wait()
            pltpu.sync_copy(acc_sh.at[pl.ds(s * 1000 + k * 40, 40)],
                            rows[b].at[pl.ds(0, 40)])
            prev[b] = pltpu.async_copy(
                rows[b].at[pl.ds(0, 40)],
                acc_out.at[c, pl.ds(s * 1000 + k * 40, 40)], sem_wb)
        prev[0].wait()
        prev[1].wait()
        pltpu.sync_copy(den_sh.at[pl.ds(s * 1000, 1000)], zden_v)
        pltpu.sync_copy(zden_v, den_out.at[pl.ds(c * N + s * 1000, 1000)])


def kernel(x, edge_index, W0, a_src0, a_dst0, b0, W1, a_src1, a_dst1, b1,
           W2, a_src2, a_dst2, b2):
    ei = edge_index.astype(jnp.int32).reshape(2, NW * NCHUNK, CH)
    ei = ei.transpose(1, 0, 2)

    r = lambda v: v.reshape(1, D)

    h, asad = _tc_first(x, W0, r(a_src0), r(a_dst0))
    acc, den = _sc_edges(ei, asad.reshape(2 * N), h)

    h, asad = _tc_mid(acc, den.reshape(NC, N, 1), r(b0), W1, r(a_src1), r(a_dst1))
    acc, den = _sc_edges(ei, asad.reshape(2 * N), h)

    h, asad = _tc_mid(acc, den.reshape(NC, N, 1), r(b1), W2, r(a_src2), r(a_dst2))
    acc, den = _sc_edges(ei, asad.reshape(2 * N), h)

    return _tc_final(acc, den.reshape(NC, N, 1), r(b2))
